# baseline (speedup 1.0000x reference)
import functools

import jax
import jax.numpy as jnp
from jax import lax
from jax.experimental import pallas as pl
from jax.experimental.pallas import tpu as pltpu
from jax.experimental.pallas import tpu_sc as plsc

CH = 55
CPAD = 64
P = 128
G = P // 16
NW = 32
OUTW = P * CH
def _tec_kernel(nchunks, table, ptsf, consts, out,
                idx_v, w_v, rows_v, p_v, c_v, out_v,
                sem0, sem1, osem0, osem1):
    wid = lax.axis_index("s") * 2 + lax.axis_index("c")
    nt = (nchunks - wid + NW - 1) // NW
    pltpu.sync_copy(consts, c_v)
    sems = (sem0, sem1)

    def fire(b, t):
        cid = wid + t * NW
        base = cid * P
        pltpu.sync_copy(ptsf.at[0, pl.ds(base, P)], p_v.at[0])
        pltpu.sync_copy(ptsf.at[1, pl.ds(base, P)], p_v.at[1])
        pltpu.sync_copy(ptsf.at[2, pl.ds(base, P)], p_v.at[2])

        def grp_index(g, c2):
            sl16 = pl.ds(g * 16, 16)
            xv = p_v[0, sl16]
            yv = p_v[1, sl16]
            zv = p_v[2, sl16]
            cd = jnp.clip(xv * c_v[0, :] + c_v[3, :], 0.0, 63.0)
            chh = jnp.clip(yv * c_v[1, :] + c_v[4, :], 0.0, 63.0)
            cw = jnp.clip(zv * c_v[2, :] + c_v[5, :], 0.0, 63.0)
            d0 = cd.astype(jnp.int32)
            h0 = chh.astype(jnp.int32)
            w0 = cw.astype(jnp.int32)
            fd = cd - d0.astype(jnp.float32)
            fh = chh - h0.astype(jnp.float32)
            fw = cw - w0.astype(jnp.float32)
            one = jnp.float32(1.0)
            gd, gh, gw = one - fd, one - fh, one - fw
            d1 = jnp.minimum(d0 + 1, 63)
            h1 = jnp.minimum(h0 + 1, 63)
            w1 = jnp.minimum(w0 + 1, 63)
            bd0 = d0 * 4096
            bd1 = d1 * 4096
            bh0 = h0 * 64
            bh1 = h1 * 64
            i00 = bd0 + bh0
            i01 = bd0 + bh1
            i10 = bd1 + bh0
            i11 = bd1 + bh1
            sl = pl.ds(g * 16, 16)
            idx_v[b, 0, sl] = i00 + w0
            idx_v[b, 1, sl] = i00 + w1
            idx_v[b, 2, sl] = i01 + w0
            idx_v[b, 3, sl] = i01 + w1
            idx_v[b, 4, sl] = i10 + w0
            idx_v[b, 5, sl] = i10 + w1
            idx_v[b, 6, sl] = i11 + w0
            idx_v[b, 7, sl] = i11 + w1
            hgw = gh * gw
            hgf = gh * fw
            hfg = fh * gw
            hff = fh * fw
            w_v[b, 0, sl] = gd * hgw
            w_v[b, 1, sl] = gd * hgf
            w_v[b, 2, sl] = gd * hfg
            w_v[b, 3, sl] = gd * hff
            w_v[b, 4, sl] = fd * hgw
            w_v[b, 5, sl] = fd * hgf
            w_v[b, 6, sl] = fd * hfg
            w_v[b, 7, sl] = fd * hff
            return c2

        lax.fori_loop(0, G, grp_index, 0)
        for j in range(8):
            pltpu.async_copy(table.at[idx_v.at[b, j]], rows_v.at[b, j],
                             sems[b])

    def wait_rows(b):
        for j in range(8):
            pltpu.make_async_copy(table.at[pl.ds(0, P)], rows_v.at[b, j],
                                  sems[b]).wait()

    osems = (osem0, osem1)

    def drain_out(b):
        pltpu.make_async_copy(out_v.at[b, pl.ds(0, OUTW)],
                              out.at[pl.ds(0, OUTW)], osems[b]).wait()

    himask = jnp.full((16,), -65536, dtype=jnp.int32)

    def blend(b, t):
        def grp_blend(g, c2):
            wrows = [w_v[b, j, pl.ds(g * 16, 16)] for j in range(8)]
            for p in range(16):
                pt = g * 16 + p
                acc = [None] * 4
                for j in range(8):
                    wsp = jnp.broadcast_to(wrows[j][p], (16,))
                    w0i = rows_v[b, j, pt, pl.ds(0, 16)]
                    w1i = rows_v[b, j, pt, pl.ds(16, 16)]
                    corner = (
                        lax.bitcast_convert_type(
                            lax.shift_left(w0i, 16), jnp.float32),
                        lax.bitcast_convert_type(w0i & himask, jnp.float32),
                        lax.bitcast_convert_type(
                            lax.shift_left(w1i, 16), jnp.float32),
                        lax.bitcast_convert_type(w1i & himask, jnp.float32),
                    )
                    for k in range(4):
                        if acc[k] is None:
                            acc[k] = wsp * corner[k]
                        else:
                            acc[k] = acc[k] + wsp * corner[k]
                for k in range(4):
                    out_v[b, pl.ds(pt * CH + 16 * k, 16)] = acc[k]
            return c2

        lax.fori_loop(0, G, grp_blend, 0)
        cid = wid + t * NW
        pltpu.async_copy(out_v.at[b, pl.ds(0, OUTW)],
                         out.at[pl.ds(cid * OUTW, OUTW)], osems[b])

    fire(0, 0)

    def body(tt, carry):
        t0 = tt * 2

        @pl.when(t0 + 1 < nt)
        def _():
            fire(1, t0 + 1)

        @pl.when(t0 >= 2)
        def _():
            drain_out(0)

        wait_rows(0)
        blend(0, t0)

        @pl.when(t0 + 1 < nt)
        def _():
            @pl.when(t0 + 2 < nt)
            def _():
                fire(0, t0 + 2)

            @pl.when(t0 >= 1)
            def _():
                drain_out(1)

            wait_rows(1)
            blend(1, t0 + 1)

        return carry

    lax.fori_loop(0, (nt + 1) // 2, body, 0)
    drain_out(0)
    drain_out(1)


@functools.partial(jax.jit, static_argnums=(3,))
def _run(table, ptsf, consts, npts):
    nchunks = npts // P
    mesh = plsc.VectorSubcoreMesh(core_axis_name="c", subcore_axis_name="s")
    kern = functools.partial(
        pl.kernel,
        out_type=jax.ShapeDtypeStruct((npts * CH,), jnp.float32),
        mesh=mesh,
        compiler_params=pltpu.CompilerParams(use_tc_tiling_on_sc=False),
        scratch_types=[
            pltpu.VMEM((2, 8, P), jnp.int32),
            pltpu.VMEM((2, 8, P), jnp.float32),
            pltpu.VMEM((2, 8, P, CPAD // 2), jnp.int32),
            pltpu.VMEM((3, P), jnp.float32),
            pltpu.VMEM((6, 16), jnp.float32),
            pltpu.VMEM((2, OUTW + 16), jnp.float32),
            pltpu.SemaphoreType.DMA,
            pltpu.SemaphoreType.DMA,
            pltpu.SemaphoreType.DMA,
            pltpu.SemaphoreType.DMA,
        ],
    )(functools.partial(_tec_kernel, nchunks))
    return kern(table, ptsf, consts)


def kernel(pts, weight_volume, volume_bounds):
    B, N, _ = pts.shape
    vol = weight_volume[0]
    C, D, H, W = vol.shape
    volp = jnp.pad(vol.reshape(C, -1), ((0, CPAD - C), (0, 0)))
    v32 = lax.bitcast_convert_type(volp, jnp.uint32)
    r = (v32 + jnp.uint32(0x7FFF) + ((v32 >> 16) & jnp.uint32(1))) >> 16
    lo = jnp.concatenate([r[0:16], r[32:48]], axis=0)
    hi = jnp.concatenate([r[16:32], r[48:64]], axis=0)
    word = (hi << 16) | lo
    table = lax.bitcast_convert_type(word.T, jnp.int32)
    ptsf = pts.reshape(-1, 3).T
    vb0 = volume_bounds[0]
    vlen = volume_bounds[1] - volume_bounds[0]
    dims = jnp.array([D - 1, H - 1, W - 1], dtype=jnp.float32)
    scale = dims / vlen
    off = -vb0 * scale
    consts = jnp.broadcast_to(
        jnp.concatenate([scale, off])[:, None], (6, 16)).astype(jnp.float32)
    out = _run(table, ptsf, consts, B * N)
    return out.reshape(B, N, CH)


# speedup vs baseline: 1.2380x; 1.2380x over previous
import functools

import jax
import jax.numpy as jnp
from jax import lax
from jax.experimental import pallas as pl
from jax.experimental.pallas import tpu as pltpu
from jax.experimental.pallas import tpu_sc as plsc

CH = 55
CPAD = 64
P = 128
G = P // 16
NW = 32
OUTW = P * CH
def _tec_kernel(nchunks, table, ptsf, consts, out,
                idx_v, w_v, rows_v, p_v, c_v, out_v,
                sem0, sem1, osem0, osem1):
    wid = lax.axis_index("s") * 2 + lax.axis_index("c")
    nt = (nchunks - wid + NW - 1) // NW
    pltpu.sync_copy(consts, c_v)
    sems = (sem0, sem1)

    def fire(b, t):
        cid = wid + t * NW
        base = cid * P
        pltpu.sync_copy(ptsf.at[0, pl.ds(base, P)], p_v.at[0])
        pltpu.sync_copy(ptsf.at[1, pl.ds(base, P)], p_v.at[1])
        pltpu.sync_copy(ptsf.at[2, pl.ds(base, P)], p_v.at[2])

        def grp_index(g, c2):
            sl16 = pl.ds(g * 16, 16)
            xv = p_v[0, sl16]
            yv = p_v[1, sl16]
            zv = p_v[2, sl16]
            cd = jnp.clip(xv * c_v[0, :] + c_v[3, :], 0.0, 63.0)
            chh = jnp.clip(yv * c_v[1, :] + c_v[4, :], 0.0, 63.0)
            cw = jnp.clip(zv * c_v[2, :] + c_v[5, :], 0.0, 63.0)
            d0 = cd.astype(jnp.int32)
            h0 = chh.astype(jnp.int32)
            w0 = cw.astype(jnp.int32)
            fd = cd - d0.astype(jnp.float32)
            fh = chh - h0.astype(jnp.float32)
            fw = cw - w0.astype(jnp.float32)
            one = jnp.float32(1.0)
            gd, gh, gw = one - fd, one - fh, one - fw
            d1 = jnp.minimum(d0 + 1, 63)
            h1 = jnp.minimum(h0 + 1, 63)
            w1 = jnp.minimum(w0 + 1, 63)
            bd0 = d0 * 4096
            bd1 = d1 * 4096
            bh0 = h0 * 64
            bh1 = h1 * 64
            i00 = bd0 + bh0
            i01 = bd0 + bh1
            i10 = bd1 + bh0
            i11 = bd1 + bh1
            sl = pl.ds(g * 16, 16)
            idx_v[b, 0, sl] = i00 + w0
            idx_v[b, 1, sl] = i00 + w1
            idx_v[b, 2, sl] = i01 + w0
            idx_v[b, 3, sl] = i01 + w1
            idx_v[b, 4, sl] = i10 + w0
            idx_v[b, 5, sl] = i10 + w1
            idx_v[b, 6, sl] = i11 + w0
            idx_v[b, 7, sl] = i11 + w1
            hgw = gh * gw
            hgf = gh * fw
            hfg = fh * gw
            hff = fh * fw
            w_v[b, 0, sl] = gd * hgw
            w_v[b, 1, sl] = gd * hgf
            w_v[b, 2, sl] = gd * hfg
            w_v[b, 3, sl] = gd * hff
            w_v[b, 4, sl] = fd * hgw
            w_v[b, 5, sl] = fd * hgf
            w_v[b, 6, sl] = fd * hfg
            w_v[b, 7, sl] = fd * hff
            return c2

        lax.fori_loop(0, G, grp_index, 0)
        for j in range(8):
            pltpu.async_copy(table.at[idx_v.at[b, j]], rows_v.at[b, j],
                             sems[b])

    def wait_rows(b):
        for j in range(8):
            pltpu.make_async_copy(table.at[pl.ds(0, P)], rows_v.at[b, j],
                                  sems[b]).wait()

    osems = (osem0, osem1)

    def drain_out(b):
        pltpu.make_async_copy(out_v.at[b, pl.ds(0, OUTW)],
                              out.at[pl.ds(0, OUTW)], osems[b]).wait()

    himask = jnp.full((16,), -65536, dtype=jnp.int32)

    def blend(b, t):
        def grp_blend(g, c2):
            wrows = [w_v[b, j, pl.ds(g * 16, 16)] for j in range(8)]
            for p in range(16):
                pt = g * 16 + p
                acc = [None] * 4
                for j in range(8):
                    wsp = jnp.broadcast_to(wrows[j][p], (16,))
                    w0i = rows_v[b, j, pt, pl.ds(0, 16)]
                    w1i = rows_v[b, j, pt, pl.ds(16, 16)]
                    corner = (
                        lax.bitcast_convert_type(
                            lax.shift_left(w0i, 16), jnp.float32),
                        lax.bitcast_convert_type(w0i & himask, jnp.float32),
                        lax.bitcast_convert_type(
                            lax.shift_left(w1i, 16), jnp.float32),
                        lax.bitcast_convert_type(w1i & himask, jnp.float32),
                    )
                    for k in range(4):
                        if acc[k] is None:
                            acc[k] = wsp * corner[k]
                        else:
                            acc[k] = acc[k] + wsp * corner[k]
                for k in range(4):
                    out_v[b, pl.ds(pt * CH + 16 * k, 16)] = acc[k]
            return c2

        lax.fori_loop(0, G, grp_blend, 0)
        cid = wid + t * NW
        pltpu.async_copy(out_v.at[b, pl.ds(0, OUTW)],
                         out.at[pl.ds(cid * OUTW, OUTW)], osems[b])

    fire(0, 0)

    def body(tt, carry):
        t0 = tt * 2

        @pl.when(t0 + 1 < nt)
        def _():
            fire(1, t0 + 1)

        @pl.when(t0 >= 2)
        def _():
            drain_out(0)

        wait_rows(0)
        blend(0, t0)

        @pl.when(t0 + 1 < nt)
        def _():
            @pl.when(t0 + 2 < nt)
            def _():
                fire(0, t0 + 2)

            @pl.when(t0 >= 1)
            def _():
                drain_out(1)

            wait_rows(1)
            blend(1, t0 + 1)

        return carry

    lax.fori_loop(0, (nt + 1) // 2, body, 0)
    drain_out(0)
    drain_out(1)


TBLK = 2048


def _table_tc(vol_ref, out_ref):
    x = vol_ref[...]
    z = jnp.zeros((CPAD - CH, TBLK), jnp.float32)
    xp = jnp.concatenate([x, z], axis=0)
    v = lax.bitcast_convert_type(xp, jnp.uint32)
    r = (v + jnp.uint32(0x7FFF) + ((v >> 16) & jnp.uint32(1))) >> 16
    lo = jnp.concatenate([r[0:16], r[32:48]], axis=0)
    hi = jnp.concatenate([r[16:32], r[48:64]], axis=0)
    w = (hi << 16) | lo
    out_ref[...] = lax.bitcast_convert_type(w.T, jnp.int32)


@functools.partial(jax.jit, static_argnums=(3,))
def _run(volf, ptsf, consts, npts):
    dhw = volf.shape[1]
    table = pl.pallas_call(
        _table_tc,
        grid=(dhw // TBLK,),
        in_specs=[pl.BlockSpec((CH, TBLK), lambda i: (0, i))],
        out_specs=pl.BlockSpec((TBLK, CPAD // 2), lambda i: (i, 0)),
        out_shape=jax.ShapeDtypeStruct((dhw, CPAD // 2), jnp.int32),
    )(volf)
    nchunks = npts // P
    mesh = plsc.VectorSubcoreMesh(core_axis_name="c", subcore_axis_name="s")
    kern = functools.partial(
        pl.kernel,
        out_type=jax.ShapeDtypeStruct((npts * CH,), jnp.float32),
        mesh=mesh,
        compiler_params=pltpu.CompilerParams(use_tc_tiling_on_sc=False),
        scratch_types=[
            pltpu.VMEM((2, 8, P), jnp.int32),
            pltpu.VMEM((2, 8, P), jnp.float32),
            pltpu.VMEM((2, 8, P, CPAD // 2), jnp.int32),
            pltpu.VMEM((3, P), jnp.float32),
            pltpu.VMEM((6, 16), jnp.float32),
            pltpu.VMEM((2, OUTW + 16), jnp.float32),
            pltpu.SemaphoreType.DMA,
            pltpu.SemaphoreType.DMA,
            pltpu.SemaphoreType.DMA,
            pltpu.SemaphoreType.DMA,
        ],
    )(functools.partial(_tec_kernel, nchunks))
    return kern(table, ptsf, consts)


def kernel(pts, weight_volume, volume_bounds):
    B, N, _ = pts.shape
    _, C, D, H, W = weight_volume.shape
    volf = weight_volume.reshape(C, D * H * W)
    ptsf = jnp.moveaxis(pts, -1, 0).reshape(3, -1)
    vb0 = volume_bounds[0]
    vlen = volume_bounds[1] - volume_bounds[0]
    dims = jnp.array([D - 1, H - 1, W - 1], dtype=jnp.float32)
    scale = dims / vlen
    off = -vb0 * scale
    consts = jnp.broadcast_to(
        jnp.concatenate([scale, off])[:, None], (6, 16)).astype(jnp.float32)
    out = _run(volf, ptsf, consts, B * N)
    return out.reshape(B, N, CH)


# speedup vs baseline: 1.3592x; 1.0978x over previous
import functools

import jax
import jax.numpy as jnp
from jax import lax
from jax.experimental import pallas as pl
from jax.experimental.pallas import tpu as pltpu
from jax.experimental.pallas import tpu_sc as plsc

CH = 55
CPAD = 64
P = 128
G = P // 16
NW = 32
OUTW = P * CH
def _tec_kernel(nchunks, table, ptsf, consts, out,
                idx_v, w_v, rows_v, p_v, c_v, out_v,
                sem0, sem1, osem0, osem1):
    wid = lax.axis_index("s") * 2 + lax.axis_index("c")
    nt = (nchunks - wid + NW - 1) // NW
    pltpu.sync_copy(consts, c_v)
    sems = (sem0, sem1)

    def fire(b, t):
        cid = wid + t * NW
        base = cid * P
        pltpu.sync_copy(ptsf.at[0, pl.ds(base, P)], p_v.at[0])
        pltpu.sync_copy(ptsf.at[1, pl.ds(base, P)], p_v.at[1])
        pltpu.sync_copy(ptsf.at[2, pl.ds(base, P)], p_v.at[2])

        def grp_index(g, c2):
            sl16 = pl.ds(g * 16, 16)
            xv = p_v[0, sl16]
            yv = p_v[1, sl16]
            zv = p_v[2, sl16]
            cd = jnp.clip(xv * c_v[0, :] + c_v[3, :], 0.0, 63.0)
            chh = jnp.clip(yv * c_v[1, :] + c_v[4, :], 0.0, 63.0)
            cw = jnp.clip(zv * c_v[2, :] + c_v[5, :], 0.0, 63.0)
            d0 = cd.astype(jnp.int32)
            h0 = chh.astype(jnp.int32)
            w0 = cw.astype(jnp.int32)
            fd = cd - d0.astype(jnp.float32)
            fh = chh - h0.astype(jnp.float32)
            fw = cw - w0.astype(jnp.float32)
            one = jnp.float32(1.0)
            gd, gh, gw = one - fd, one - fh, one - fw
            d1 = jnp.minimum(d0 + 1, 63)
            h1 = jnp.minimum(h0 + 1, 63)
            w1 = jnp.minimum(w0 + 1, 63)
            bd0 = d0 * 4096
            bd1 = d1 * 4096
            bh0 = h0 * 64
            bh1 = h1 * 64
            i00 = bd0 + bh0
            i01 = bd0 + bh1
            i10 = bd1 + bh0
            i11 = bd1 + bh1
            sl = pl.ds(g * 16, 16)
            idx_v[b, 0, sl] = i00 + w0
            idx_v[b, 1, sl] = i00 + w1
            idx_v[b, 2, sl] = i01 + w0
            idx_v[b, 3, sl] = i01 + w1
            idx_v[b, 4, sl] = i10 + w0
            idx_v[b, 5, sl] = i10 + w1
            idx_v[b, 6, sl] = i11 + w0
            idx_v[b, 7, sl] = i11 + w1
            hgw = gh * gw
            hgf = gh * fw
            hfg = fh * gw
            hff = fh * fw
            w_v[b, 0, sl] = gd * hgw
            w_v[b, 1, sl] = gd * hgf
            w_v[b, 2, sl] = gd * hfg
            w_v[b, 3, sl] = gd * hff
            w_v[b, 4, sl] = fd * hgw
            w_v[b, 5, sl] = fd * hgf
            w_v[b, 6, sl] = fd * hfg
            w_v[b, 7, sl] = fd * hff
            return c2

        lax.fori_loop(0, G, grp_index, 0)
        for j in range(8):
            pltpu.async_copy(table.at[idx_v.at[b, j]], rows_v.at[b, j],
                             sems[b])

    def wait_rows(b):
        for j in range(8):
            pltpu.make_async_copy(table.at[pl.ds(0, P)], rows_v.at[b, j],
                                  sems[b]).wait()

    osems = (osem0, osem1)

    def drain_out(b):
        pltpu.make_async_copy(out_v.at[b, pl.ds(0, OUTW)],
                              out.at[pl.ds(0, OUTW)], osems[b]).wait()

    himask = jnp.full((16,), -65536, dtype=jnp.int32)

    def blend(b, t):
        def grp_blend(g, c2):
            wrows = [w_v[b, j, pl.ds(g * 16, 16)] for j in range(8)]
            for p in range(16):
                pt = g * 16 + p
                acc = [None] * 4
                for j in range(8):
                    wsp = jnp.broadcast_to(wrows[j][p], (16,))
                    w0i = rows_v[b, j, pt, pl.ds(0, 16)]
                    w1i = rows_v[b, j, pt, pl.ds(16, 16)]
                    corner = (
                        lax.bitcast_convert_type(
                            lax.shift_left(w0i, 16), jnp.float32),
                        lax.bitcast_convert_type(w0i & himask, jnp.float32),
                        lax.bitcast_convert_type(
                            lax.shift_left(w1i, 16), jnp.float32),
                        lax.bitcast_convert_type(w1i & himask, jnp.float32),
                    )
                    for k in range(4):
                        if acc[k] is None:
                            acc[k] = wsp * corner[k]
                        else:
                            acc[k] = acc[k] + wsp * corner[k]
                for k in range(4):
                    out_v[b, pl.ds(pt * CH + 16 * k, 16)] = acc[k]
            return c2

        lax.fori_loop(0, G, grp_blend, 0)
        cid = wid + t * NW
        pltpu.async_copy(out_v.at[b, pl.ds(0, OUTW)],
                         out.at[pl.ds(cid * OUTW, OUTW)], osems[b])

    fire(0, 0)

    def body(tt, carry):
        t0 = tt * 2

        @pl.when(t0 + 1 < nt)
        def _():
            fire(1, t0 + 1)

        @pl.when(t0 >= 2)
        def _():
            drain_out(0)

        wait_rows(0)
        blend(0, t0)

        @pl.when(t0 + 1 < nt)
        def _():
            @pl.when(t0 + 2 < nt)
            def _():
                fire(0, t0 + 2)

            @pl.when(t0 >= 1)
            def _():
                drain_out(1)

            wait_rows(1)
            blend(1, t0 + 1)

        return carry

    lax.fori_loop(0, (nt + 1) // 2, body, 0)
    drain_out(0)
    drain_out(1)


TBLK = 4096


def _table_tc(vol_ref, out_ref):
    x = vol_ref[0, :, 0].reshape(CH, TBLK)
    z = jnp.zeros((CPAD - CH, TBLK), jnp.float32)
    xp = jnp.concatenate([x, z], axis=0)
    v = lax.bitcast_convert_type(xp, jnp.uint32)
    r = (v + jnp.uint32(0x7FFF) + ((v >> 16) & jnp.uint32(1))) >> 16
    lo = jnp.concatenate([r[0:16], r[32:48]], axis=0)
    hi = jnp.concatenate([r[16:32], r[48:64]], axis=0)
    w = (hi << 16) | lo
    out_ref[...] = lax.bitcast_convert_type(w.T, jnp.int32)


@functools.partial(jax.jit, static_argnums=(3,))
def _run(vol5, ptsf, consts, npts):
    d_dim = vol5.shape[2]
    dhw = d_dim * vol5.shape[3] * vol5.shape[4]
    table = pl.pallas_call(
        _table_tc,
        grid=(d_dim,),
        in_specs=[pl.BlockSpec((1, CH, 1, 64, 64), lambda i: (0, 0, i, 0, 0))],
        out_specs=pl.BlockSpec((TBLK, CPAD // 2), lambda i: (i, 0)),
        out_shape=jax.ShapeDtypeStruct((dhw, CPAD // 2), jnp.int32),
    )(vol5)
    nchunks = npts // P
    mesh = plsc.VectorSubcoreMesh(core_axis_name="c", subcore_axis_name="s")
    kern = functools.partial(
        pl.kernel,
        out_type=jax.ShapeDtypeStruct((npts * CH,), jnp.float32),
        mesh=mesh,
        compiler_params=pltpu.CompilerParams(use_tc_tiling_on_sc=False),
        scratch_types=[
            pltpu.VMEM((2, 8, P), jnp.int32),
            pltpu.VMEM((2, 8, P), jnp.float32),
            pltpu.VMEM((2, 8, P, CPAD // 2), jnp.int32),
            pltpu.VMEM((3, P), jnp.float32),
            pltpu.VMEM((6, 16), jnp.float32),
            pltpu.VMEM((2, OUTW + 16), jnp.float32),
            pltpu.SemaphoreType.DMA,
            pltpu.SemaphoreType.DMA,
            pltpu.SemaphoreType.DMA,
            pltpu.SemaphoreType.DMA,
        ],
    )(functools.partial(_tec_kernel, nchunks))
    return kern(table, ptsf, consts)


def kernel(pts, weight_volume, volume_bounds):
    B, N, _ = pts.shape
    _, C, D, H, W = weight_volume.shape
    ptsf = jnp.moveaxis(pts, -1, 0).reshape(3, -1)
    vb0 = volume_bounds[0]
    vlen = volume_bounds[1] - volume_bounds[0]
    dims = jnp.array([D - 1, H - 1, W - 1], dtype=jnp.float32)
    scale = dims / vlen
    off = -vb0 * scale
    consts = jnp.broadcast_to(
        jnp.concatenate([scale, off])[:, None], (6, 16)).astype(jnp.float32)
    out = _run(weight_volume, ptsf, consts, B * N)
    return out.reshape(B, N, CH)


# speedup vs baseline: 1.6170x; 1.1897x over previous
import functools

import jax
import jax.numpy as jnp
from jax import lax
from jax.experimental import pallas as pl
from jax.experimental.pallas import tpu as pltpu
from jax.experimental.pallas import tpu_sc as plsc

CH = 55
CPAD = 64
P = 128
G = P // 16
NW = 32
OUTW = P * CH
def _tec_kernel(nchunks, table, ptsf, consts, out,
                idx_v, w_v, rows_v, p_v, c_v, out_v,
                sem0, sem1, osem0, osem1, psem):
    wid = lax.axis_index("s") * 2 + lax.axis_index("c")
    nt = (nchunks - wid + NW - 1) // NW
    pltpu.sync_copy(consts, c_v)
    sems = (sem0, sem1)

    def pts_load(b, t):
        cid = jnp.minimum(wid + t * NW, nchunks - 1)
        base = cid * P
        pltpu.async_copy(ptsf.at[:, pl.ds(base, P)], p_v.at[b], psem)

    def pts_drain(b):
        pltpu.make_async_copy(ptsf.at[:, pl.ds(0, P)], p_v.at[b],
                              psem).wait()

    def fire(b, t):
        pts_load(1 - b, t + 1)
        pts_drain(b)

        def grp_index(g, c2):
            sl16 = pl.ds(g * 16, 16)
            xv = p_v[b, 0, sl16]
            yv = p_v[b, 1, sl16]
            zv = p_v[b, 2, sl16]
            cd = jnp.clip(xv * c_v[0, :] + c_v[3, :], 0.0, 63.0)
            chh = jnp.clip(yv * c_v[1, :] + c_v[4, :], 0.0, 63.0)
            cw = jnp.clip(zv * c_v[2, :] + c_v[5, :], 0.0, 63.0)
            d0 = cd.astype(jnp.int32)
            h0 = chh.astype(jnp.int32)
            w0 = cw.astype(jnp.int32)
            fd = cd - d0.astype(jnp.float32)
            fh = chh - h0.astype(jnp.float32)
            fw = cw - w0.astype(jnp.float32)
            one = jnp.float32(1.0)
            gd, gh, gw = one - fd, one - fh, one - fw
            d1 = jnp.minimum(d0 + 1, 63)
            h1 = jnp.minimum(h0 + 1, 63)
            w1 = jnp.minimum(w0 + 1, 63)
            bd0 = d0 * 4096
            bd1 = d1 * 4096
            bh0 = h0 * 64
            bh1 = h1 * 64
            i00 = bd0 + bh0
            i01 = bd0 + bh1
            i10 = bd1 + bh0
            i11 = bd1 + bh1
            sl = pl.ds(g * 16, 16)
            idx_v[b, 0, sl] = i00 + w0
            idx_v[b, 1, sl] = i00 + w1
            idx_v[b, 2, sl] = i01 + w0
            idx_v[b, 3, sl] = i01 + w1
            idx_v[b, 4, sl] = i10 + w0
            idx_v[b, 5, sl] = i10 + w1
            idx_v[b, 6, sl] = i11 + w0
            idx_v[b, 7, sl] = i11 + w1
            hgw = gh * gw
            hgf = gh * fw
            hfg = fh * gw
            hff = fh * fw
            w_v[b, 0, sl] = gd * hgw
            w_v[b, 1, sl] = gd * hgf
            w_v[b, 2, sl] = gd * hfg
            w_v[b, 3, sl] = gd * hff
            w_v[b, 4, sl] = fd * hgw
            w_v[b, 5, sl] = fd * hgf
            w_v[b, 6, sl] = fd * hfg
            w_v[b, 7, sl] = fd * hff
            return c2

        lax.fori_loop(0, G, grp_index, 0)
        for j in range(8):
            pltpu.async_copy(table.at[idx_v.at[b, j]], rows_v.at[b, j],
                             sems[b])

    def wait_rows(b):
        for j in range(8):
            pltpu.make_async_copy(table.at[pl.ds(0, P)], rows_v.at[b, j],
                                  sems[b]).wait()

    osems = (osem0, osem1)

    def drain_out(b):
        pltpu.make_async_copy(out_v.at[b, pl.ds(0, OUTW)],
                              out.at[pl.ds(0, OUTW)], osems[b]).wait()

    def blend(b, t):
        def grp_blend(g, c2):
            wrows = [w_v[b, j, pl.ds(g * 16, 16)] for j in range(8)]
            for p in range(16):
                pt = g * 16 + p
                acc = [None] * 4
                for j in range(8):
                    wsp = jnp.broadcast_to(wrows[j][p], (16,))
                    w0i = rows_v[b, j, pt, pl.ds(0, 16)]
                    w1i = rows_v[b, j, pt, pl.ds(16, 16)]
                    corner = (
                        lax.bitcast_convert_type(
                            lax.shift_left(w0i, 16), jnp.float32),
                        lax.bitcast_convert_type(w0i, jnp.float32),
                        lax.bitcast_convert_type(
                            lax.shift_left(w1i, 16), jnp.float32),
                        lax.bitcast_convert_type(w1i, jnp.float32),
                    )
                    for k in range(4):
                        if acc[k] is None:
                            acc[k] = wsp * corner[k]
                        else:
                            acc[k] = acc[k] + wsp * corner[k]
                for k in range(4):
                    out_v[b, pl.ds(pt * CH + 16 * k, 16)] = acc[k]
            return c2

        lax.fori_loop(0, G, grp_blend, 0)
        cid = wid + t * NW
        pltpu.async_copy(out_v.at[b, pl.ds(0, OUTW)],
                         out.at[pl.ds(cid * OUTW, OUTW)], osems[b])

    pts_load(0, 0)
    fire(0, 0)

    def body(tt, carry):
        t0 = tt * 2

        @pl.when(t0 + 1 < nt)
        def _():
            fire(1, t0 + 1)

        @pl.when(t0 >= 2)
        def _():
            drain_out(0)

        wait_rows(0)
        blend(0, t0)

        @pl.when(t0 + 1 < nt)
        def _():
            @pl.when(t0 + 2 < nt)
            def _():
                fire(0, t0 + 2)

            @pl.when(t0 >= 1)
            def _():
                drain_out(1)

            wait_rows(1)
            blend(1, t0 + 1)

        return carry

    lax.fori_loop(0, (nt + 1) // 2, body, 0)
    drain_out(0)
    drain_out(1)
    pts_drain(0)


TBLK = 4096


def _table_tc(vol_ref, out_ref):
    x = vol_ref[0, :, 0].reshape(CH, TBLK)
    z = jnp.zeros((CPAD - CH, TBLK), jnp.float32)
    xp = jnp.concatenate([x, z], axis=0)
    v = lax.bitcast_convert_type(xp, jnp.uint32)
    r = (v + jnp.uint32(0x7FFF) + ((v >> 16) & jnp.uint32(1))) >> 16
    lo = jnp.concatenate([r[0:16], r[32:48]], axis=0)
    hi = jnp.concatenate([r[16:32], r[48:64]], axis=0)
    w = (hi << 16) | lo
    out_ref[...] = lax.bitcast_convert_type(w.T, jnp.int32)


@functools.partial(jax.jit, static_argnums=(3,))
def _run(vol5, ptsf, consts, npts):
    d_dim = vol5.shape[2]
    dhw = d_dim * vol5.shape[3] * vol5.shape[4]
    table = pl.pallas_call(
        _table_tc,
        grid=(d_dim,),
        in_specs=[pl.BlockSpec((1, CH, 1, 64, 64), lambda i: (0, 0, i, 0, 0))],
        out_specs=pl.BlockSpec((TBLK, CPAD // 2), lambda i: (i, 0)),
        out_shape=jax.ShapeDtypeStruct((dhw, CPAD // 2), jnp.int32),
    )(vol5)
    nchunks = npts // P
    mesh = plsc.VectorSubcoreMesh(core_axis_name="c", subcore_axis_name="s")
    kern = functools.partial(
        pl.kernel,
        out_type=jax.ShapeDtypeStruct((npts * CH,), jnp.float32),
        mesh=mesh,
        compiler_params=pltpu.CompilerParams(use_tc_tiling_on_sc=False),
        scratch_types=[
            pltpu.VMEM((2, 8, P), jnp.int32),
            pltpu.VMEM((2, 8, P), jnp.float32),
            pltpu.VMEM((2, 8, P, CPAD // 2), jnp.int32),
            pltpu.VMEM((2, 3, P), jnp.float32),
            pltpu.VMEM((6, 16), jnp.float32),
            pltpu.VMEM((2, OUTW + 16), jnp.float32),
            pltpu.SemaphoreType.DMA,
            pltpu.SemaphoreType.DMA,
            pltpu.SemaphoreType.DMA,
            pltpu.SemaphoreType.DMA,
            pltpu.SemaphoreType.DMA,
        ],
    )(functools.partial(_tec_kernel, nchunks))
    return kern(table, ptsf, consts)


def kernel(pts, weight_volume, volume_bounds):
    B, N, _ = pts.shape
    _, C, D, H, W = weight_volume.shape
    ptsf = jnp.moveaxis(pts, -1, 0).reshape(3, -1)
    vb0 = volume_bounds[0]
    vlen = volume_bounds[1] - volume_bounds[0]
    dims = jnp.array([D - 1, H - 1, W - 1], dtype=jnp.float32)
    scale = dims / vlen
    off = -vb0 * scale
    consts = jnp.broadcast_to(
        jnp.concatenate([scale, off])[:, None], (6, 16)).astype(jnp.float32)
    out = _run(weight_volume, ptsf, consts, B * N)
    return out.reshape(B, N, CH)


# speedup vs baseline: 2.6631x; 1.6470x over previous
import functools

import jax
import jax.numpy as jnp
from jax import lax
from jax.experimental import pallas as pl
from jax.experimental.pallas import tpu as pltpu
from jax.experimental.pallas import tpu_sc as plsc

CH = 55
CPAD = 64
P = 128
G = P // 16
NW = 32
OUTW = P * CH
def _tec_kernel(nchunks, table, ptsf, consts, out,
                idx_v, w_v, rows_v, p_v, c_v, out_v,
                sem0, sem1, osem0, osem1, psem):
    wid = lax.axis_index("s") * 2 + lax.axis_index("c")
    nt = (nchunks - wid + NW - 1) // NW
    pltpu.sync_copy(consts, c_v)
    sems = (sem0, sem1)

    def pts_load(b, t):
        cid = jnp.minimum(wid + t * NW, nchunks - 1)
        base = cid * P
        pltpu.async_copy(ptsf.at[:, pl.ds(base, P)], p_v.at[b], psem)

    def pts_drain(b):
        pltpu.make_async_copy(ptsf.at[:, pl.ds(0, P)], p_v.at[b],
                              psem).wait()

    def fire(b, t):
        pts_load(1 - b, t + 1)
        pts_drain(b)

        def grp_index(g, c2):
            sl16 = pl.ds(g * 16, 16)
            xv = p_v[b, 0, sl16]
            yv = p_v[b, 1, sl16]
            zv = p_v[b, 2, sl16]
            cd = jnp.clip(xv * c_v[0, :] + c_v[3, :], 0.0, 63.0)
            chh = jnp.clip(yv * c_v[1, :] + c_v[4, :], 0.0, 63.0)
            cw = jnp.clip(zv * c_v[2, :] + c_v[5, :], 0.0, 63.0)
            d0 = cd.astype(jnp.int32)
            h0 = chh.astype(jnp.int32)
            w0 = cw.astype(jnp.int32)
            fd = cd - d0.astype(jnp.float32)
            fh = chh - h0.astype(jnp.float32)
            fw = cw - w0.astype(jnp.float32)
            one = jnp.float32(1.0)
            gd, gh, gw = one - fd, one - fh, one - fw
            d1 = jnp.minimum(d0 + 1, 63)
            h1 = jnp.minimum(h0 + 1, 63)
            w1 = jnp.minimum(w0 + 1, 63)
            bd0 = d0 * 16384
            bd1 = d1 * 16384
            bh0 = h0 * 256
            bh1 = h1 * 256
            w04 = w0 * 4
            w14 = w1 * 4
            i00 = bd0 + bh0
            i01 = bd0 + bh1
            i10 = bd1 + bh0
            i11 = bd1 + bh1
            sl = pl.ds(g * 16, 16)
            idx_v[b, 0, sl] = i00 + w04
            idx_v[b, 1, sl] = i00 + w14
            idx_v[b, 2, sl] = i01 + w04
            idx_v[b, 3, sl] = i01 + w14
            idx_v[b, 4, sl] = i10 + w04
            idx_v[b, 5, sl] = i10 + w14
            idx_v[b, 6, sl] = i11 + w04
            idx_v[b, 7, sl] = i11 + w14
            hgw = gh * gw
            hgf = gh * fw
            hfg = fh * gw
            hff = fh * fw
            w_v[b, 0, sl] = gd * hgw
            w_v[b, 1, sl] = gd * hgf
            w_v[b, 2, sl] = gd * hfg
            w_v[b, 3, sl] = gd * hff
            w_v[b, 4, sl] = fd * hgw
            w_v[b, 5, sl] = fd * hgf
            w_v[b, 6, sl] = fd * hfg
            w_v[b, 7, sl] = fd * hff
            return c2

        lax.fori_loop(0, G, grp_index, 0)
        for j in range(8):
            pltpu.async_copy(table.at[idx_v.at[b, j]], rows_v.at[b, j],
                             sems[b])

    def wait_rows(b):
        for j in range(8):
            pltpu.make_async_copy(table.at[pl.ds(0, P)], rows_v.at[b, j],
                                  sems[b]).wait()

    osems = (osem0, osem1)

    def drain_out(b):
        pltpu.make_async_copy(out_v.at[b],
                              out.at[pl.ds(0, P), pl.ds(0, CPAD)],
                              osems[b]).wait()

    def blend(b, t):
        def grp_blend(g, c2):
            wrows = [w_v[b, j, pl.ds(g * 16, 16)] for j in range(8)]
            for p in range(16):
                pt = g * 16 + p
                acc = [None] * 4
                for j in range(8):
                    wsp = jnp.broadcast_to(wrows[j][p], (16,))
                    w0i = rows_v[b, j, pt, pl.ds(0, 16)]
                    w1i = rows_v[b, j, pt, pl.ds(16, 16)]
                    corner = (
                        lax.bitcast_convert_type(
                            lax.shift_left(w0i, 16), jnp.float32),
                        lax.bitcast_convert_type(w0i, jnp.float32),
                        lax.bitcast_convert_type(
                            lax.shift_left(w1i, 16), jnp.float32),
                        lax.bitcast_convert_type(w1i, jnp.float32),
                    )
                    for k in range(4):
                        if acc[k] is None:
                            acc[k] = wsp * corner[k]
                        else:
                            acc[k] = acc[k] + wsp * corner[k]
                for k in range(4):
                    out_v[b, pt, pl.ds(16 * k, 16)] = acc[k]
            return c2

        lax.fori_loop(0, G, grp_blend, 0)
        cid = wid + t * NW
        pltpu.async_copy(out_v.at[b],
                         out.at[pl.ds(cid * P, P), pl.ds(0, CPAD)],
                         osems[b])

    pts_load(0, 0)
    fire(0, 0)

    def body(tt, carry):
        t0 = tt * 2

        @pl.when(t0 + 1 < nt)
        def _():
            fire(1, t0 + 1)

        @pl.when(t0 >= 2)
        def _():
            drain_out(0)

        wait_rows(0)
        blend(0, t0)

        @pl.when(t0 + 1 < nt)
        def _():
            @pl.when(t0 + 2 < nt)
            def _():
                fire(0, t0 + 2)

            @pl.when(t0 >= 1)
            def _():
                drain_out(1)

            wait_rows(1)
            blend(1, t0 + 1)

        return carry

    lax.fori_loop(0, (nt + 1) // 2, body, 0)
    drain_out(0)
    drain_out(1)
    pts_drain(0)


TBLK = 4096


def _table_tc(vol_ref, out_ref):
    x = vol_ref[0, :, 0].reshape(CH, TBLK)
    z = jnp.zeros((CPAD - CH, TBLK), jnp.float32)
    xp = jnp.concatenate([x, z], axis=0)
    v = lax.bitcast_convert_type(xp, jnp.uint32)
    r = (v + jnp.uint32(0x7FFF) + ((v >> 16) & jnp.uint32(1))) >> 16
    lo = jnp.concatenate([r[0:16], r[32:48]], axis=0)
    hi = jnp.concatenate([r[16:32], r[48:64]], axis=0)
    w = (hi << 16) | lo
    wt = lax.bitcast_convert_type(w.T, jnp.int32)
    out_ref[...] = jnp.concatenate(
        [wt, jnp.zeros((TBLK, 96), jnp.int32)], axis=1)


@functools.partial(jax.jit, static_argnums=(3,))
def _run(vol5, ptsf, consts, npts):
    d_dim = vol5.shape[2]
    dhw = d_dim * vol5.shape[3] * vol5.shape[4]
    table = pl.pallas_call(
        _table_tc,
        grid=(d_dim,),
        in_specs=[pl.BlockSpec((1, CH, 1, 64, 64), lambda i: (0, 0, i, 0, 0))],
        out_specs=pl.BlockSpec((TBLK, 128), lambda i: (i, 0)),
        out_shape=jax.ShapeDtypeStruct((dhw, 128), jnp.int32),
    )(vol5)
    table = table.reshape(dhw * 4, CPAD // 2)
    nchunks = npts // P
    mesh = plsc.VectorSubcoreMesh(core_axis_name="c", subcore_axis_name="s")
    kern = functools.partial(
        pl.kernel,
        out_type=jax.ShapeDtypeStruct((npts, 128), jnp.float32),
        mesh=mesh,
        compiler_params=pltpu.CompilerParams(use_tc_tiling_on_sc=False),
        scratch_types=[
            pltpu.VMEM((2, 8, P), jnp.int32),
            pltpu.VMEM((2, 8, P), jnp.float32),
            pltpu.VMEM((2, 8, P, CPAD // 2), jnp.int32),
            pltpu.VMEM((2, 3, P), jnp.float32),
            pltpu.VMEM((6, 16), jnp.float32),
            pltpu.VMEM((2, P, CPAD), jnp.float32),
            pltpu.SemaphoreType.DMA,
            pltpu.SemaphoreType.DMA,
            pltpu.SemaphoreType.DMA,
            pltpu.SemaphoreType.DMA,
            pltpu.SemaphoreType.DMA,
        ],
    )(functools.partial(_tec_kernel, nchunks))
    return kern(table, ptsf, consts)


def kernel(pts, weight_volume, volume_bounds):
    B, N, _ = pts.shape
    _, C, D, H, W = weight_volume.shape
    ptsf = jnp.moveaxis(pts, -1, 0).reshape(3, -1)
    vb0 = volume_bounds[0]
    vlen = volume_bounds[1] - volume_bounds[0]
    dims = jnp.array([D - 1, H - 1, W - 1], dtype=jnp.float32)
    scale = dims / vlen
    off = -vb0 * scale
    consts = jnp.broadcast_to(
        jnp.concatenate([scale, off])[:, None], (6, 16)).astype(jnp.float32)
    out = _run(weight_volume, ptsf, consts, B * N)
    return out.reshape(B, N, 128)[..., :CH]
